# Initial kernel scaffold; baseline (speedup 1.0000x reference)
import jax
import jax.numpy as jnp
from jax.experimental import pallas as pl


def kernel(points, inverse_map, voxel_coors, pre_gamma, pre_beta, W0, bn0_gamma, bn0_beta, W1, b1, Wc, bc):
    raise NotImplementedError("write your pallas kernel here")


# speedup vs baseline: 1.6803x; 1.6803x over previous
import functools

import jax
import jax.numpy as jnp
from jax import lax
from jax.experimental import pallas as pl
from jax.experimental.pallas import tpu as pltpu
from jax.experimental.pallas import tpu_sc as plsc

N = 1_600_000
V = 100_000
EPS = 1e-5
NC, NS, LANES = 2, 16, 16
NW = NC * NS
P = N // NW
TRASH = V
VP = V + 8
NEGINF = float("-inf")

CH = 80
GG = 2000
RB = 400
FB = 128
VROWS = V // NS


def _mesh():
  return plsc.VectorSubcoreMesh(core_axis_name="c", subcore_axis_name="s")


def _lanes_i32():
  return lax.broadcasted_iota(jnp.int32, (LANES,), 0)


def _sc_scatter_body(points_hbm, ids_hbm, ones_hbm, zeros_hbm, out_hbm,
                     vals_v, idx_v, ones_v, shared_xyz, shared_cnt):
  c = lax.axis_index("c")
  s = lax.axis_index("s")
  wid = c * NS + s
  base = wid * P

  pltpu.sync_copy(zeros_hbm.at[pl.ds(s * VROWS, VROWS)],
                  shared_xyz.at[pl.ds(s * VROWS, VROWS)])
  pltpu.sync_copy(zeros_hbm.at[pl.ds(s * VROWS, VROWS)],
                  shared_cnt.at[pl.ds(s * VROWS, VROWS)])
  pltpu.sync_copy(ones_hbm, ones_v)
  plsc.subcore_barrier()

  def chunk(j, _):
    off = base + j * CH
    pltpu.sync_copy(ids_hbm.at[pl.ds(off, CH)], idx_v)
    pltpu.sync_copy(points_hbm.at[pl.ds(off, CH)], vals_v)
    pltpu.sync_copy(vals_v, shared_xyz.at[idx_v], add=True)
    pltpu.sync_copy(ones_v, shared_cnt.at[idx_v], add=True)
    return _

  lax.fori_loop(0, P // CH, chunk, 0)
  plsc.subcore_barrier()

  pltpu.sync_copy(shared_xyz.at[pl.ds(s * VROWS, VROWS)],
                  out_hbm.at[c, 0, pl.ds(s * VROWS, VROWS)])
  pltpu.sync_copy(shared_cnt.at[pl.ds(s * VROWS, VROWS)],
                  out_hbm.at[c, 1, pl.ds(s * VROWS, VROWS)])


def _sc_scatter(points, inverse_map, ones_rows, zeros_rows):
  k = functools.partial(
      pl.kernel,
      out_type=jax.ShapeDtypeStruct((NC, 2, V, 4), jnp.float32),
      mesh=_mesh(),
      compiler_params=pltpu.CompilerParams(use_tc_tiling_on_sc=False),
      scratch_types=[
          pltpu.VMEM((CH, 4), jnp.float32),
          pltpu.VMEM((CH,), jnp.int32),
          pltpu.VMEM((CH, 4), jnp.float32),
          pltpu.VMEM_SHARED((V, 4), jnp.float32),
          pltpu.VMEM_SHARED((V, 4), jnp.float32),
      ],
  )(_sc_scatter_body)
  return k(points, inverse_map, ones_rows, zeros_rows)


def _t1_body(parts_ref, out_ref):
  x = parts_ref[...]
  col = lax.broadcasted_iota(jnp.int32, (1, x.shape[1]), 1) % 4
  out_ref[...] = jnp.where(col == 3, x[1:2] + x[3:4], x[0:1] + x[2:3])


def _t1_combine(parts_flat):
  BL = 80_000
  return pl.pallas_call(
      _t1_body,
      grid=(5,),
      in_specs=[pl.BlockSpec((4, BL), lambda i: (0, i))],
      out_specs=pl.BlockSpec((1, BL), lambda i: (0, i)),
      out_shape=jax.ShapeDtypeStruct((1, 4 * V), jnp.float32),
  )(parts_flat)


def _sc_gather_body(table_hbm, ids_hbm, out_hbm, idx_v, rows_v):
  c = lax.axis_index("c")
  s = lax.axis_index("s")
  base = (c * NS + s) * P

  def chunk(j, _):
    off = base + j * GG
    pltpu.sync_copy(ids_hbm.at[pl.ds(off, GG)], idx_v)
    pltpu.sync_copy(table_hbm.at[idx_v], rows_v)
    pltpu.sync_copy(rows_v, out_hbm.at[pl.ds(off, GG)])
    return _

  lax.fori_loop(0, P // GG, chunk, 0)


def _sc_gather(table, inverse_map):
  k = functools.partial(
      pl.kernel,
      out_type=jax.ShapeDtypeStruct((N, 4), jnp.float32),
      mesh=_mesh(),
      compiler_params=pltpu.CompilerParams(use_tc_tiling_on_sc=False),
      scratch_types=[
          pltpu.VMEM((GG,), jnp.int32),
          pltpu.VMEM((GG, 4), jnp.float32),
      ],
  )(_sc_gather_body)
  return k(table, inverse_map)


BS_STATS = 4_000


def _features(p, srows):
  cnt = srows[:, 3:4]
  m3 = srows[:, 0:3] / cnt
  d = jnp.sqrt(jnp.sum(p[:, 0:3] * p[:, 0:3], axis=1, keepdims=True))
  return jnp.concatenate([p, d, p[:, 0:3] - m3], axis=1)


def _stats_body(p_ref, s_ref, out_ref):
  f = _features(p_ref[...], s_ref[...])
  m2 = lax.dot_general(f, f, (((0,), (0,)), ((), ())),
                       preferred_element_type=jnp.float32)
  contrib = jnp.concatenate([jnp.sum(f, axis=0, keepdims=True), m2], axis=0)
  pid = pl.program_id(0)

  @pl.when(pid == 0)
  def _():
    out_ref[...] = contrib

  @pl.when(pid > 0)
  def _():
    out_ref[...] += contrib


def _tc_stats(points, srows):
  return pl.pallas_call(
      _stats_body,
      grid=(N // BS_STATS,),
      in_specs=[
          pl.BlockSpec((BS_STATS, 4), lambda i: (i, 0)),
          pl.BlockSpec((BS_STATS, 4), lambda i: (i, 0)),
      ],
      out_specs=pl.BlockSpec((9, 8), lambda i: (0, 0)),
      out_shape=jax.ShapeDtypeStruct((9, 8), jnp.float32),
      compiler_params=pltpu.CompilerParams(
          dimension_semantics=("arbitrary",)),
  )(points, srows)


BD = 4_000


def _main_body(p_ref, s_ref, weff_ref, beff_ref, w1_ref, b1_ref,
               h0_ref, h1_ref):
  f = _features(p_ref[...], s_ref[...])
  h0 = lax.dot_general(f, weff_ref[...], (((1,), (1,)), ((), ())),
                       preferred_element_type=jnp.float32)
  h0 = jnp.maximum(h0 + beff_ref[...], 0.0)
  h0_ref[...] = h0
  h1 = lax.dot_general(h0, w1_ref[...], (((1,), (1,)), ((), ())),
                       preferred_element_type=jnp.float32)
  h1_ref[...] = h1 + b1_ref[...]


def _tc_main(points, srows, weff, beff, w1, b1):
  return pl.pallas_call(
      _main_body,
      grid=(N // BD,),
      in_specs=[
          pl.BlockSpec((BD, 4), lambda i: (i, 0)),
          pl.BlockSpec((BD, 4), lambda i: (i, 0)),
          pl.BlockSpec((64, 8), lambda i: (0, 0)),
          pl.BlockSpec((1, 64), lambda i: (0, 0)),
          pl.BlockSpec((128, 64), lambda i: (0, 0)),
          pl.BlockSpec((1, 128), lambda i: (0, 0)),
      ],
      out_specs=[
          pl.BlockSpec((BD, 64), lambda i: (i, 0)),
          pl.BlockSpec((BD, 128), lambda i: (i, 0)),
      ],
      out_shape=[
          jax.ShapeDtypeStruct((N, 64), jnp.float32),
          jax.ShapeDtypeStruct((N, 128), jnp.float32),
      ],
      compiler_params=pltpu.CompilerParams(
          dimension_semantics=("arbitrary",)),
  )(points, srows, weff, beff, w1, b1)


def _row_copy(dst_ref, di, src_ref, si):
  for kk in range(8):
    dst_ref[di, pl.ds(kk * 16, 16)] = src_ref[si, pl.ds(kk * 16, 16)]


def _reset_batch(accbuf, fidx, nrows):
  neg = jnp.full((LANES,), NEGINF, jnp.float32)

  def zrow(i, _):
    for kk in range(8):
      accbuf[i, pl.ds(kk * 16, 16)] = neg
    return _

  lax.fori_loop(0, nrows, zrow, 0)
  tr = jnp.full((LANES,), TRASH, jnp.int32)
  for q in range(nrows // 16):
    fidx[pl.ds(q * 16, 16)] = tr


def _sc_segmax_body(h1_hbm, ids_hbm, vmax_hbm, seamr_hbm, seami_hbm,
                    rows_v, ids_v, accbuf, fidx, headrow, tailrow,
                    comb_rows, comb_acc, comb_idx, cids_v, idstage,
                    pair_v, sidebuf, sideids):
  c = lax.axis_index("c")
  s = lax.axis_index("s")
  wid = c * NS + s
  a = wid * P
  lanes = _lanes_i32()

  pltpu.sync_copy(ids_hbm.at[pl.ds(a, 8)], idstage.at[pl.ds(0, 8)])

  @pl.when(wid < NW - 1)
  def _():
    pltpu.sync_copy(ids_hbm.at[pl.ds(a + P, 8)], idstage.at[pl.ds(8, 8)])

  idvec0 = idstage[...]
  first_id = idvec0[0]
  next_first = jnp.where(wid == NW - 1, V, idvec0[8])
  start_row = jnp.where(wid == 0, 0, first_id)

  _reset_batch(accbuf, fidx, FB)

  span = next_first - start_row
  nfull = span // FB
  nrem = span - nfull * FB

  def init_body(i, _):
    pltpu.sync_copy(accbuf, vmax_hbm.at[pl.ds(start_row + i * FB, FB)])
    return _

  lax.fori_loop(0, nfull, init_body, 0)
  rem_base = start_row + nfull * FB

  def init1_body(i, _):
    pltpu.sync_copy(accbuf.at[pl.ds(0, 1)], vmax_hbm.at[pl.ds(rem_base + i, 1)])
    return _

  lax.fori_loop(0, nrem, init1_body, 0)

  def flush(nflush):
    pltpu.sync_copy(accbuf, vmax_hbm.at[fidx])

    @pl.when(nflush == 0)
    def _():
      _row_copy(headrow, 0, accbuf, 0)

    _reset_batch(accbuf, fidx, FB)

  def point(i, carry):
    prev, slot, nflush = carry
    pid = ids_v[pl.ds(i, 16)][0]
    new = pid != prev
    do_flush = jnp.logical_and(new, slot == FB - 1)

    @pl.when(do_flush)
    def _():
      flush(nflush)

    nflush2 = jnp.where(do_flush, nflush + 1, nflush)
    slot2 = jnp.where(new, jnp.where(do_flush, 0, slot + 1), slot)

    @pl.when(new)
    def _():
      is_head = jnp.logical_and(nflush2 == 0, slot2 == 0)
      off = (slot2 // 16) * 16
      vv = fidx[pl.ds(off, 16)]
      val = jnp.where(is_head, TRASH, pid)
      fidx[pl.ds(off, 16)] = jnp.where(lanes + off == slot2, val, vv)

    for kk in range(8):
      av = accbuf[slot2, pl.ds(kk * 16, 16)]
      rv = rows_v[i, pl.ds(kk * 16, 16)]
      accbuf[slot2, pl.ds(kk * 16, 16)] = jnp.maximum(av, rv)

    return pid, slot2, nflush2

  def block(j, carry):
    off = a + j * RB
    pltpu.sync_copy(h1_hbm.at[pl.ds(off, RB)], rows_v)
    pltpu.sync_copy(ids_hbm.at[pl.ds(off, RB)], ids_v.at[pl.ds(0, RB)])
    return lax.fori_loop(0, RB, point, carry)

  prev, slot, nflush = lax.fori_loop(
      0, P // RB, block, (jnp.int32(-1), jnp.int32(-1), jnp.int32(0)))

  _row_copy(tailrow, 0, accbuf, slot)

  @pl.when(nflush == 0)
  def _():
    _row_copy(headrow, 0, accbuf, 0)

  off = (slot // 16) * 16
  vv = fidx[pl.ds(off, 16)]
  fidx[pl.ds(off, 16)] = jnp.where(lanes + off == slot, TRASH, vv)
  pltpu.sync_copy(accbuf, vmax_hbm.at[fidx])

  _row_copy(pair_v, 0, headrow, 0)
  _row_copy(pair_v, 1, tailrow, 0)
  pltpu.sync_copy(pair_v, sidebuf.at[pl.ds(2 * s, 2)])
  idvec = jnp.where(lanes == 0, first_id,
                    jnp.where(lanes == 1, prev, TRASH)).astype(jnp.int32)
  idstage[...] = idvec
  pltpu.sync_copy(idstage, sideids.at[s])
  plsc.subcore_barrier()

  @pl.when(s == 0)
  def _():
    pltpu.sync_copy(sidebuf, comb_rows)
    pltpu.sync_copy(sideids, cids_v)
    _reset_batch(comb_acc, comb_idx, 32)

    def centry(e, carry):
      prev2, slot2 = carry
      crow = cids_v[e // 2, pl.ds(0, 16)]
      eid = jnp.where(e % 2 == 0, crow[0], crow[1])
      new = eid != prev2
      slot3 = jnp.where(new, slot2 + 1, slot2)

      @pl.when(new)
      def _():
        is_excl = jnp.logical_and(c == 1, slot3 == 0)
        off2 = (slot3 // 16) * 16
        vv2 = comb_idx[pl.ds(off2, 16)]
        val2 = jnp.where(is_excl, TRASH, eid)
        comb_idx[pl.ds(off2, 16)] = jnp.where(
            lanes + off2 == slot3, val2, vv2)

      for kk in range(8):
        av = comb_acc[slot3, pl.ds(kk * 16, 16)]
        rv = comb_rows[e, pl.ds(kk * 16, 16)]
        comb_acc[slot3, pl.ds(kk * 16, 16)] = jnp.maximum(av, rv)

      return eid, slot3

    last_id, last_slot = lax.fori_loop(
        0, 2 * NS, centry, (jnp.int32(-1), jnp.int32(-1)))

    seam_slot = jnp.where(c == 0, last_slot, 0)
    seam_id = jnp.where(c == 0, last_id, cids_v[0, pl.ds(0, 16)][0])

    @pl.when(c == 0)
    def _():
      off3 = (last_slot // 16) * 16
      vv3 = comb_idx[pl.ds(off3, 16)]
      comb_idx[pl.ds(off3, 16)] = jnp.where(
          lanes + off3 == last_slot, TRASH, vv3)

    _row_copy(pair_v, 0, comb_acc, seam_slot)
    pltpu.sync_copy(comb_acc, vmax_hbm.at[comb_idx])
    pltpu.sync_copy(pair_v.at[pl.ds(0, 1)], seamr_hbm.at[pl.ds(c, 1)])
    idvec2 = jnp.where(lanes == 0, seam_id, TRASH).astype(jnp.int32)
    idstage[...] = idvec2
    pltpu.sync_copy(idstage.at[pl.ds(0, 8)], seami_hbm.at[c])


def _sc_segmax(h1, inverse_map):
  k = functools.partial(
      pl.kernel,
      out_type=(
          jax.ShapeDtypeStruct((VP, 128), jnp.float32),
          jax.ShapeDtypeStruct((2, 128), jnp.float32),
          jax.ShapeDtypeStruct((2, 8), jnp.int32),
      ),
      mesh=_mesh(),
      compiler_params=pltpu.CompilerParams(use_tc_tiling_on_sc=False),
      scratch_types=[
          pltpu.VMEM((RB, 128), jnp.float32),
          pltpu.VMEM((RB + 16,), jnp.int32),
          pltpu.VMEM((FB, 128), jnp.float32),
          pltpu.VMEM((FB,), jnp.int32),
          pltpu.VMEM((1, 128), jnp.float32),
          pltpu.VMEM((1, 128), jnp.float32),
          pltpu.VMEM((32, 128), jnp.float32),
          pltpu.VMEM((32, 128), jnp.float32),
          pltpu.VMEM((32,), jnp.int32),
          pltpu.VMEM((16, 16), jnp.int32),
          pltpu.VMEM((16,), jnp.int32),
          pltpu.VMEM((2, 128), jnp.float32),
          pltpu.VMEM_SHARED((32, 128), jnp.float32),
          pltpu.VMEM_SHARED((16, 16), jnp.int32),
      ],
  )(_sc_segmax_body)
  return k(h1, inverse_map)


VB = 5_000


def _compress_body(vmax_ref, seamr_ref, seami_ref, wc_ref, bc_ref, out_ref):
  blk = vmax_ref[...]
  seam_rows = seamr_ref[...]
  seam_ids = seami_ref[...]
  vlo = pl.program_id(0) * VB
  rows = lax.broadcasted_iota(jnp.int32, (VB, 1), 0)
  for k in range(2):
    rel = seam_ids[k, 0] - vlo
    blk = jnp.where(rows == rel,
                    jnp.maximum(blk, seam_rows[k:k + 1, :]), blk)
  out = lax.dot_general(blk, wc_ref[...], (((1,), (1,)), ((), ())),
                        preferred_element_type=jnp.float32)
  out_ref[...] = jnp.maximum(out + bc_ref[...], 0.0)


def _tc_compress(vmax_padded, seam_rows, seam_ids, wc, bc):
  return pl.pallas_call(
      _compress_body,
      grid=(V // VB,),
      in_specs=[
          pl.BlockSpec((VB, 128), lambda i: (i, 0)),
          pl.BlockSpec((2, 128), lambda i: (0, 0)),
          pl.BlockSpec((2, 8), lambda i: (0, 0)),
          pl.BlockSpec((16, 128), lambda i: (0, 0)),
          pl.BlockSpec((1, 16), lambda i: (0, 0)),
      ],
      out_specs=pl.BlockSpec((VB, 16), lambda i: (i, 0)),
      out_shape=jax.ShapeDtypeStruct((V, 16), jnp.float32),
  )(vmax_padded, seam_rows, seam_ids, wc, bc)


def kernel(points, inverse_map, voxel_coors, pre_gamma, pre_beta, W0,
           bn0_gamma, bn0_beta, W1, b1, Wc, bc):
  del voxel_coors
  inverse_map = inverse_map.astype(jnp.int32)
  ones_rows = jnp.ones((CH, 4), jnp.float32)
  zeros_rows = jnp.zeros((V, 4), jnp.float32)

  parts = _sc_scatter(points, inverse_map, ones_rows, zeros_rows)
  s_flat = _t1_combine(parts.reshape(4, 4 * V))
  s_table = s_flat.reshape(V, 4)
  srows = _sc_gather(s_table, inverse_map)

  mom = _tc_stats(points, srows)
  sum_f = mom[0]
  m2 = mom[1:9]
  mu = sum_f / N
  cov = m2 / N - mu[:, None] * mu[None, :]
  var_f = jnp.diagonal(cov)
  a = pre_gamma / jnp.sqrt(var_f + EPS)
  cvec = pre_beta - mu * a
  mean0 = W0 @ pre_beta
  aw = W0 * a[None, :]
  var0 = jnp.sum((aw @ cov) * aw, axis=1)
  s0 = bn0_gamma / jnp.sqrt(var0 + EPS)
  t0 = bn0_beta - mean0 * s0
  weff = aw * s0[:, None]
  beff = s0 * (W0 @ cvec) + t0

  h0, h1 = _tc_main(points, srows, weff, beff.reshape(1, 64), W1,
                    b1.reshape(1, 128))

  vmax_padded, seam_rows, seam_ids = _sc_segmax(h1, inverse_map)
  vf = _tc_compress(vmax_padded, seam_rows, seam_ids, Wc, bc.reshape(1, 16))
  return vf, h0, h1


# speedup vs baseline: 1.7841x; 1.0618x over previous
import functools

import jax
import jax.numpy as jnp
from jax import lax
from jax.experimental import pallas as pl
from jax.experimental.pallas import tpu as pltpu
from jax.experimental.pallas import tpu_sc as plsc

N = 1_600_000
V = 100_000
EPS = 1e-5
NC, NS, LANES = 2, 16, 16
NW = NC * NS
P = N // NW
TRASH = V
VP = V + 8
NEGINF = float("-inf")

CH = 80
GG = 2000
RB = 400
FB = 128
VROWS = V // NS


def _mesh():
  return plsc.VectorSubcoreMesh(core_axis_name="c", subcore_axis_name="s")


def _lanes_i32():
  return lax.broadcasted_iota(jnp.int32, (LANES,), 0)


def _sc_scatter_body(points_hbm, ids_hbm, ones_hbm, zeros_hbm, out_hbm,
                     vals_v, idx_v, ones_v, shared_xyz, shared_cnt):
  c = lax.axis_index("c")
  s = lax.axis_index("s")
  wid = c * NS + s
  base = wid * P

  pltpu.sync_copy(zeros_hbm.at[pl.ds(s * VROWS, VROWS)],
                  shared_xyz.at[pl.ds(s * VROWS, VROWS)])
  pltpu.sync_copy(zeros_hbm.at[pl.ds(s * VROWS, VROWS)],
                  shared_cnt.at[pl.ds(s * VROWS, VROWS)])
  pltpu.sync_copy(ones_hbm, ones_v)
  plsc.subcore_barrier()

  def chunk(j, _):
    off = base + j * CH
    pltpu.sync_copy(ids_hbm.at[pl.ds(off, CH)], idx_v)
    pltpu.sync_copy(points_hbm.at[pl.ds(off, CH)], vals_v)
    pltpu.sync_copy(vals_v, shared_xyz.at[idx_v], add=True)
    pltpu.sync_copy(ones_v, shared_cnt.at[idx_v], add=True)
    return _

  lax.fori_loop(0, P // CH, chunk, 0)
  plsc.subcore_barrier()

  pltpu.sync_copy(shared_xyz.at[pl.ds(s * VROWS, VROWS)],
                  out_hbm.at[c, 0, pl.ds(s * VROWS, VROWS)])
  pltpu.sync_copy(shared_cnt.at[pl.ds(s * VROWS, VROWS)],
                  out_hbm.at[c, 1, pl.ds(s * VROWS, VROWS)])


def _sc_scatter(points, inverse_map, ones_rows, zeros_rows):
  k = functools.partial(
      pl.kernel,
      out_type=jax.ShapeDtypeStruct((NC, 2, V, 4), jnp.float32),
      mesh=_mesh(),
      compiler_params=pltpu.CompilerParams(use_tc_tiling_on_sc=False),
      scratch_types=[
          pltpu.VMEM((CH, 4), jnp.float32),
          pltpu.VMEM((CH,), jnp.int32),
          pltpu.VMEM((CH, 4), jnp.float32),
          pltpu.VMEM_SHARED((V, 4), jnp.float32),
          pltpu.VMEM_SHARED((V, 4), jnp.float32),
      ],
  )(_sc_scatter_body)
  return k(points, inverse_map, ones_rows, zeros_rows)


def _t1_body(parts_ref, out_ref):
  x = parts_ref[...]
  col = lax.broadcasted_iota(jnp.int32, (1, x.shape[1]), 1) % 4
  out_ref[...] = jnp.where(col == 3, x[1:2] + x[3:4], x[0:1] + x[2:3])


def _t1_combine(parts_flat):
  BL = 80_000
  return pl.pallas_call(
      _t1_body,
      grid=(5,),
      in_specs=[pl.BlockSpec((4, BL), lambda i: (0, i))],
      out_specs=pl.BlockSpec((1, BL), lambda i: (0, i)),
      out_shape=jax.ShapeDtypeStruct((1, 4 * V), jnp.float32),
  )(parts_flat)


def _sc_gather_body(table_hbm, ids_hbm, out_hbm, idx_v, rows_v):
  c = lax.axis_index("c")
  s = lax.axis_index("s")
  base = (c * NS + s) * P

  def chunk(j, _):
    off = base + j * GG
    pltpu.sync_copy(ids_hbm.at[pl.ds(off, GG)], idx_v)
    pltpu.sync_copy(table_hbm.at[idx_v], rows_v)
    pltpu.sync_copy(rows_v, out_hbm.at[pl.ds(off, GG)])
    return _

  lax.fori_loop(0, P // GG, chunk, 0)


def _sc_gather(table, inverse_map):
  k = functools.partial(
      pl.kernel,
      out_type=jax.ShapeDtypeStruct((N, 4), jnp.float32),
      mesh=_mesh(),
      compiler_params=pltpu.CompilerParams(use_tc_tiling_on_sc=False),
      scratch_types=[
          pltpu.VMEM((GG,), jnp.int32),
          pltpu.VMEM((GG, 4), jnp.float32),
      ],
  )(_sc_gather_body)
  return k(table, inverse_map)


BS_STATS = 8_000


def _features(p, srows):
  cnt = srows[:, 3:4]
  m3 = srows[:, 0:3] / cnt
  d = jnp.sqrt(jnp.sum(p[:, 0:3] * p[:, 0:3], axis=1, keepdims=True))
  return jnp.concatenate([p, d, p[:, 0:3] - m3], axis=1)


def _stats_body(p_ref, s_ref, out_ref):
  f = _features(p_ref[...], s_ref[...])
  m2 = lax.dot_general(f, f, (((0,), (0,)), ((), ())),
                       preferred_element_type=jnp.float32)
  contrib = jnp.concatenate([jnp.sum(f, axis=0, keepdims=True), m2], axis=0)
  pid = pl.program_id(0)

  @pl.when(pid == 0)
  def _():
    out_ref[...] = contrib

  @pl.when(pid > 0)
  def _():
    out_ref[...] += contrib


def _tc_stats(points, srows):
  return pl.pallas_call(
      _stats_body,
      grid=(N // BS_STATS,),
      in_specs=[
          pl.BlockSpec((BS_STATS, 4), lambda i: (i, 0)),
          pl.BlockSpec((BS_STATS, 4), lambda i: (i, 0)),
      ],
      out_specs=pl.BlockSpec((9, 8), lambda i: (0, 0)),
      out_shape=jax.ShapeDtypeStruct((9, 8), jnp.float32),
      compiler_params=pltpu.CompilerParams(
          dimension_semantics=("arbitrary",)),
  )(points, srows)


BD = 8_000


def _main_body(p_ref, s_ref, weff_ref, beff_ref, w1_ref, b1_ref,
               h0_ref, h1_ref):
  f = _features(p_ref[...], s_ref[...])
  h0 = lax.dot_general(f, weff_ref[...], (((1,), (1,)), ((), ())),
                       preferred_element_type=jnp.float32)
  h0 = jnp.maximum(h0 + beff_ref[...], 0.0)
  h0_ref[...] = h0
  h1 = lax.dot_general(h0, w1_ref[...], (((1,), (1,)), ((), ())),
                       preferred_element_type=jnp.float32)
  h1_ref[...] = h1 + b1_ref[...]


def _tc_main(points, srows, weff, beff, w1, b1):
  return pl.pallas_call(
      _main_body,
      grid=(N // BD,),
      in_specs=[
          pl.BlockSpec((BD, 4), lambda i: (i, 0)),
          pl.BlockSpec((BD, 4), lambda i: (i, 0)),
          pl.BlockSpec((64, 8), lambda i: (0, 0)),
          pl.BlockSpec((1, 64), lambda i: (0, 0)),
          pl.BlockSpec((128, 64), lambda i: (0, 0)),
          pl.BlockSpec((1, 128), lambda i: (0, 0)),
      ],
      out_specs=[
          pl.BlockSpec((BD, 64), lambda i: (i, 0)),
          pl.BlockSpec((BD, 128), lambda i: (i, 0)),
      ],
      out_shape=[
          jax.ShapeDtypeStruct((N, 64), jnp.float32),
          jax.ShapeDtypeStruct((N, 128), jnp.float32),
      ],
      compiler_params=pltpu.CompilerParams(
          dimension_semantics=("arbitrary",)),
  )(points, srows, weff, beff, w1, b1)


def _row_copy(dst_ref, di, src_ref, si):
  for kk in range(8):
    dst_ref[di, pl.ds(kk * 16, 16)] = src_ref[si, pl.ds(kk * 16, 16)]


def _reset_batch(accbuf, fidx, nrows):
  neg = jnp.full((LANES,), NEGINF, jnp.float32)

  def zrow(i, _):
    for kk in range(8):
      accbuf[i, pl.ds(kk * 16, 16)] = neg
    return _

  lax.fori_loop(0, nrows, zrow, 0)
  tr = jnp.full((LANES,), TRASH, jnp.int32)
  for q in range(nrows // 16):
    fidx[pl.ds(q * 16, 16)] = tr


def _sc_segmax_body(h1_hbm, ids_hbm, vmax_hbm, seamr_hbm, seami_hbm,
                    rows_v, ids_v, accbuf, fidx, headrow, tailrow,
                    comb_rows, comb_acc, comb_idx, cids_v, idstage,
                    pair_v, sidebuf, sideids):
  c = lax.axis_index("c")
  s = lax.axis_index("s")
  wid = c * NS + s
  a = wid * P
  lanes = _lanes_i32()

  pltpu.sync_copy(ids_hbm.at[pl.ds(a, 8)], idstage.at[pl.ds(0, 8)])

  @pl.when(wid < NW - 1)
  def _():
    pltpu.sync_copy(ids_hbm.at[pl.ds(a + P, 8)], idstage.at[pl.ds(8, 8)])

  idvec0 = idstage[...]
  first_id = idvec0[0]
  next_first = jnp.where(wid == NW - 1, V, idvec0[8])
  start_row = jnp.where(wid == 0, 0, first_id)

  _reset_batch(accbuf, fidx, FB)

  span = next_first - start_row
  nfull = span // FB
  nrem = span - nfull * FB

  def init_body(i, _):
    pltpu.sync_copy(accbuf, vmax_hbm.at[pl.ds(start_row + i * FB, FB)])
    return _

  lax.fori_loop(0, nfull, init_body, 0)
  rem_base = start_row + nfull * FB

  def init1_body(i, _):
    pltpu.sync_copy(accbuf.at[pl.ds(0, 1)], vmax_hbm.at[pl.ds(rem_base + i, 1)])
    return _

  lax.fori_loop(0, nrem, init1_body, 0)

  def flush(nflush):
    pltpu.sync_copy(accbuf, vmax_hbm.at[fidx])

    @pl.when(nflush == 0)
    def _():
      _row_copy(headrow, 0, accbuf, 0)

    _reset_batch(accbuf, fidx, FB)

  def point(i, carry):
    prev, slot, nflush = carry
    pid = ids_v[pl.ds(i, 16)][0]
    new = pid != prev
    do_flush = jnp.logical_and(new, slot == FB - 1)

    @pl.when(do_flush)
    def _():
      flush(nflush)

    nflush2 = jnp.where(do_flush, nflush + 1, nflush)
    slot2 = jnp.where(new, jnp.where(do_flush, 0, slot + 1), slot)

    @pl.when(new)
    def _():
      is_head = jnp.logical_and(nflush2 == 0, slot2 == 0)
      off = (slot2 // 16) * 16
      vv = fidx[pl.ds(off, 16)]
      val = jnp.where(is_head, TRASH, pid)
      fidx[pl.ds(off, 16)] = jnp.where(lanes + off == slot2, val, vv)

    for kk in range(8):
      av = accbuf[slot2, pl.ds(kk * 16, 16)]
      rv = rows_v[i, pl.ds(kk * 16, 16)]
      accbuf[slot2, pl.ds(kk * 16, 16)] = jnp.maximum(av, rv)

    return pid, slot2, nflush2

  def block(j, carry):
    off = a + j * RB
    pltpu.sync_copy(h1_hbm.at[pl.ds(off, RB)], rows_v)
    pltpu.sync_copy(ids_hbm.at[pl.ds(off, RB)], ids_v.at[pl.ds(0, RB)])
    return lax.fori_loop(0, RB, point, carry)

  prev, slot, nflush = lax.fori_loop(
      0, P // RB, block, (jnp.int32(-1), jnp.int32(-1), jnp.int32(0)))

  _row_copy(tailrow, 0, accbuf, slot)

  @pl.when(nflush == 0)
  def _():
    _row_copy(headrow, 0, accbuf, 0)

  off = (slot // 16) * 16
  vv = fidx[pl.ds(off, 16)]
  fidx[pl.ds(off, 16)] = jnp.where(lanes + off == slot, TRASH, vv)
  pltpu.sync_copy(accbuf, vmax_hbm.at[fidx])

  _row_copy(pair_v, 0, headrow, 0)
  _row_copy(pair_v, 1, tailrow, 0)
  pltpu.sync_copy(pair_v, sidebuf.at[pl.ds(2 * s, 2)])
  idvec = jnp.where(lanes == 0, first_id,
                    jnp.where(lanes == 1, prev, TRASH)).astype(jnp.int32)
  idstage[...] = idvec
  pltpu.sync_copy(idstage, sideids.at[s])
  plsc.subcore_barrier()

  @pl.when(s == 0)
  def _():
    pltpu.sync_copy(sidebuf, comb_rows)
    pltpu.sync_copy(sideids, cids_v)
    _reset_batch(comb_acc, comb_idx, 32)

    def centry(e, carry):
      prev2, slot2 = carry
      crow = cids_v[e // 2, pl.ds(0, 16)]
      eid = jnp.where(e % 2 == 0, crow[0], crow[1])
      new = eid != prev2
      slot3 = jnp.where(new, slot2 + 1, slot2)

      @pl.when(new)
      def _():
        is_excl = jnp.logical_and(c == 1, slot3 == 0)
        off2 = (slot3 // 16) * 16
        vv2 = comb_idx[pl.ds(off2, 16)]
        val2 = jnp.where(is_excl, TRASH, eid)
        comb_idx[pl.ds(off2, 16)] = jnp.where(
            lanes + off2 == slot3, val2, vv2)

      for kk in range(8):
        av = comb_acc[slot3, pl.ds(kk * 16, 16)]
        rv = comb_rows[e, pl.ds(kk * 16, 16)]
        comb_acc[slot3, pl.ds(kk * 16, 16)] = jnp.maximum(av, rv)

      return eid, slot3

    last_id, last_slot = lax.fori_loop(
        0, 2 * NS, centry, (jnp.int32(-1), jnp.int32(-1)))

    seam_slot = jnp.where(c == 0, last_slot, 0)
    seam_id = jnp.where(c == 0, last_id, cids_v[0, pl.ds(0, 16)][0])

    @pl.when(c == 0)
    def _():
      off3 = (last_slot // 16) * 16
      vv3 = comb_idx[pl.ds(off3, 16)]
      comb_idx[pl.ds(off3, 16)] = jnp.where(
          lanes + off3 == last_slot, TRASH, vv3)

    _row_copy(pair_v, 0, comb_acc, seam_slot)
    pltpu.sync_copy(comb_acc, vmax_hbm.at[comb_idx])
    pltpu.sync_copy(pair_v.at[pl.ds(0, 1)], seamr_hbm.at[pl.ds(c, 1)])
    idvec2 = jnp.where(lanes == 0, seam_id, TRASH).astype(jnp.int32)
    idstage[...] = idvec2
    pltpu.sync_copy(idstage.at[pl.ds(0, 8)], seami_hbm.at[c])


def _sc_segmax(h1, inverse_map):
  k = functools.partial(
      pl.kernel,
      out_type=(
          jax.ShapeDtypeStruct((VP, 128), jnp.float32),
          jax.ShapeDtypeStruct((2, 128), jnp.float32),
          jax.ShapeDtypeStruct((2, 8), jnp.int32),
      ),
      mesh=_mesh(),
      compiler_params=pltpu.CompilerParams(use_tc_tiling_on_sc=False),
      scratch_types=[
          pltpu.VMEM((RB, 128), jnp.float32),
          pltpu.VMEM((RB + 16,), jnp.int32),
          pltpu.VMEM((FB, 128), jnp.float32),
          pltpu.VMEM((FB,), jnp.int32),
          pltpu.VMEM((1, 128), jnp.float32),
          pltpu.VMEM((1, 128), jnp.float32),
          pltpu.VMEM((32, 128), jnp.float32),
          pltpu.VMEM((32, 128), jnp.float32),
          pltpu.VMEM((32,), jnp.int32),
          pltpu.VMEM((16, 16), jnp.int32),
          pltpu.VMEM((16,), jnp.int32),
          pltpu.VMEM((2, 128), jnp.float32),
          pltpu.VMEM_SHARED((32, 128), jnp.float32),
          pltpu.VMEM_SHARED((16, 16), jnp.int32),
      ],
  )(_sc_segmax_body)
  return k(h1, inverse_map)


VB = 5_000


def _compress_body(vmax_ref, seamr_ref, seami_ref, wc_ref, bc_ref, out_ref):
  blk = vmax_ref[...]
  seam_rows = seamr_ref[...]
  seam_ids = seami_ref[...]
  vlo = pl.program_id(0) * VB
  rows = lax.broadcasted_iota(jnp.int32, (VB, 1), 0)
  for k in range(2):
    rel = seam_ids[k, 0] - vlo
    blk = jnp.where(rows == rel,
                    jnp.maximum(blk, seam_rows[k:k + 1, :]), blk)
  out = lax.dot_general(blk, wc_ref[...], (((1,), (1,)), ((), ())),
                        preferred_element_type=jnp.float32)
  out_ref[...] = jnp.maximum(out + bc_ref[...], 0.0)


def _tc_compress(vmax_padded, seam_rows, seam_ids, wc, bc):
  return pl.pallas_call(
      _compress_body,
      grid=(V // VB,),
      in_specs=[
          pl.BlockSpec((VB, 128), lambda i: (i, 0)),
          pl.BlockSpec((2, 128), lambda i: (0, 0)),
          pl.BlockSpec((2, 8), lambda i: (0, 0)),
          pl.BlockSpec((16, 128), lambda i: (0, 0)),
          pl.BlockSpec((1, 16), lambda i: (0, 0)),
      ],
      out_specs=pl.BlockSpec((VB, 16), lambda i: (i, 0)),
      out_shape=jax.ShapeDtypeStruct((V, 16), jnp.float32),
  )(vmax_padded, seam_rows, seam_ids, wc, bc)


def kernel(points, inverse_map, voxel_coors, pre_gamma, pre_beta, W0,
           bn0_gamma, bn0_beta, W1, b1, Wc, bc):
  del voxel_coors
  inverse_map = inverse_map.astype(jnp.int32)
  ones_rows = jnp.ones((CH, 4), jnp.float32)
  zeros_rows = jnp.zeros((V, 4), jnp.float32)

  parts = _sc_scatter(points, inverse_map, ones_rows, zeros_rows)
  s_flat = _t1_combine(parts.reshape(4, 4 * V))
  s_table = s_flat.reshape(V, 4)
  srows = _sc_gather(s_table, inverse_map)

  mom = _tc_stats(points, srows)
  sum_f = mom[0]
  m2 = mom[1:9]
  mu = sum_f / N
  cov = m2 / N - mu[:, None] * mu[None, :]
  var_f = jnp.diagonal(cov)
  a = pre_gamma / jnp.sqrt(var_f + EPS)
  cvec = pre_beta - mu * a
  mean0 = W0 @ pre_beta
  aw = W0 * a[None, :]
  var0 = jnp.sum((aw @ cov) * aw, axis=1)
  s0 = bn0_gamma / jnp.sqrt(var0 + EPS)
  t0 = bn0_beta - mean0 * s0
  weff = aw * s0[:, None]
  beff = s0 * (W0 @ cvec) + t0

  h0, h1 = _tc_main(points, srows, weff, beff.reshape(1, 64), W1,
                    b1.reshape(1, 128))

  vmax_padded, seam_rows, seam_ids = _sc_segmax(h1, inverse_map)
  vf = _tc_compress(vmax_padded, seam_rows, seam_ids, Wc, bc.reshape(1, 16))
  return vf, h0, h1


# speedup vs baseline: 1.7873x; 1.0018x over previous
import functools

import jax
import jax.numpy as jnp
from jax import lax
from jax.experimental import pallas as pl
from jax.experimental.pallas import tpu as pltpu
from jax.experimental.pallas import tpu_sc as plsc

N = 1_600_000
V = 100_000
EPS = 1e-5
NC, NS, LANES = 2, 16, 16
NW = NC * NS
P = N // NW
TRASH = V
VP = V + 8
NEGINF = float("-inf")

CH = 80
GG = 2000
RB = 400
FB = 128
VROWS = V // NS


def _mesh():
  return plsc.VectorSubcoreMesh(core_axis_name="c", subcore_axis_name="s")


def _lanes_i32():
  return lax.broadcasted_iota(jnp.int32, (LANES,), 0)


def _sc_scatter_body(points_hbm, ids_hbm, ones_hbm, zeros_hbm, out_hbm,
                     vals_v, idx_v, ones_v, shared_xyz, shared_cnt):
  c = lax.axis_index("c")
  s = lax.axis_index("s")
  wid = c * NS + s
  base = wid * P

  pltpu.sync_copy(zeros_hbm.at[pl.ds(s * VROWS, VROWS)],
                  shared_xyz.at[pl.ds(s * VROWS, VROWS)])
  pltpu.sync_copy(zeros_hbm.at[pl.ds(s * VROWS, VROWS)],
                  shared_cnt.at[pl.ds(s * VROWS, VROWS)])
  pltpu.sync_copy(ones_hbm, ones_v)
  plsc.subcore_barrier()

  def chunk(j, _):
    off = base + j * CH
    pltpu.sync_copy(ids_hbm.at[pl.ds(off, CH)], idx_v)
    pltpu.sync_copy(points_hbm.at[pl.ds(off, CH)], vals_v)
    pltpu.sync_copy(vals_v, shared_xyz.at[idx_v], add=True)
    pltpu.sync_copy(ones_v, shared_cnt.at[idx_v], add=True)
    return _

  lax.fori_loop(0, P // CH, chunk, 0)
  plsc.subcore_barrier()

  pltpu.sync_copy(shared_xyz.at[pl.ds(s * VROWS, VROWS)],
                  out_hbm.at[c, 0, pl.ds(s * VROWS, VROWS)])
  pltpu.sync_copy(shared_cnt.at[pl.ds(s * VROWS, VROWS)],
                  out_hbm.at[c, 1, pl.ds(s * VROWS, VROWS)])


def _sc_scatter(points, inverse_map, ones_rows, zeros_rows):
  k = functools.partial(
      pl.kernel,
      out_type=jax.ShapeDtypeStruct((NC, 2, V, 4), jnp.float32),
      mesh=_mesh(),
      compiler_params=pltpu.CompilerParams(use_tc_tiling_on_sc=False),
      scratch_types=[
          pltpu.VMEM((CH, 4), jnp.float32),
          pltpu.VMEM((CH,), jnp.int32),
          pltpu.VMEM((CH, 4), jnp.float32),
          pltpu.VMEM_SHARED((V, 4), jnp.float32),
          pltpu.VMEM_SHARED((V, 4), jnp.float32),
      ],
  )(_sc_scatter_body)
  return k(points, inverse_map, ones_rows, zeros_rows)


def _t1_body(parts_ref, out_ref):
  x = parts_ref[...]
  col = lax.broadcasted_iota(jnp.int32, (1, x.shape[1]), 1) % 4
  out_ref[...] = jnp.where(col == 3, x[1:2] + x[3:4], x[0:1] + x[2:3])


def _t1_combine(parts_flat):
  BL = 80_000
  return pl.pallas_call(
      _t1_body,
      grid=(5,),
      in_specs=[pl.BlockSpec((4, BL), lambda i: (0, i))],
      out_specs=pl.BlockSpec((1, BL), lambda i: (0, i)),
      out_shape=jax.ShapeDtypeStruct((1, 4 * V), jnp.float32),
  )(parts_flat)


def _sc_gather_body(table_hbm, ids_hbm, out_hbm, idx_v, rows_v):
  c = lax.axis_index("c")
  s = lax.axis_index("s")
  base = (c * NS + s) * P

  def chunk(j, _):
    off = base + j * GG
    pltpu.sync_copy(ids_hbm.at[pl.ds(off, GG)], idx_v)
    pltpu.sync_copy(table_hbm.at[idx_v], rows_v)
    pltpu.sync_copy(rows_v, out_hbm.at[pl.ds(off, GG)])
    return _

  lax.fori_loop(0, P // GG, chunk, 0)


def _sc_gather(table, inverse_map):
  k = functools.partial(
      pl.kernel,
      out_type=jax.ShapeDtypeStruct((N, 4), jnp.float32),
      mesh=_mesh(),
      compiler_params=pltpu.CompilerParams(use_tc_tiling_on_sc=False),
      scratch_types=[
          pltpu.VMEM((GG,), jnp.int32),
          pltpu.VMEM((GG, 4), jnp.float32),
      ],
  )(_sc_gather_body)
  return k(table, inverse_map)


BS_STATS = 16_000


def _features(p, srows):
  cnt = srows[:, 3:4]
  m3 = srows[:, 0:3] / cnt
  d = jnp.sqrt(jnp.sum(p[:, 0:3] * p[:, 0:3], axis=1, keepdims=True))
  return jnp.concatenate([p, d, p[:, 0:3] - m3], axis=1)


def _stats_body(p_ref, s_ref, out_ref):
  f = _features(p_ref[...], s_ref[...])
  m2 = lax.dot_general(f, f, (((0,), (0,)), ((), ())),
                       preferred_element_type=jnp.float32)
  contrib = jnp.concatenate([jnp.sum(f, axis=0, keepdims=True), m2], axis=0)
  pid = pl.program_id(0)

  @pl.when(pid == 0)
  def _():
    out_ref[...] = contrib

  @pl.when(pid > 0)
  def _():
    out_ref[...] += contrib


def _tc_stats(points, srows):
  return pl.pallas_call(
      _stats_body,
      grid=(N // BS_STATS,),
      in_specs=[
          pl.BlockSpec((BS_STATS, 4), lambda i: (i, 0)),
          pl.BlockSpec((BS_STATS, 4), lambda i: (i, 0)),
      ],
      out_specs=pl.BlockSpec((9, 8), lambda i: (0, 0)),
      out_shape=jax.ShapeDtypeStruct((9, 8), jnp.float32),
      compiler_params=pltpu.CompilerParams(
          dimension_semantics=("arbitrary",)),
  )(points, srows)


BD = 8_000


def _main_body(p_ref, s_ref, weff_ref, beff_ref, w1_ref, b1_ref,
               h0_ref, h1_ref):
  f = _features(p_ref[...], s_ref[...])
  h0 = lax.dot_general(f, weff_ref[...], (((1,), (1,)), ((), ())),
                       preferred_element_type=jnp.float32)
  h0 = jnp.maximum(h0 + beff_ref[...], 0.0)
  h0_ref[...] = h0
  h1 = lax.dot_general(h0, w1_ref[...], (((1,), (1,)), ((), ())),
                       preferred_element_type=jnp.float32)
  h1_ref[...] = h1 + b1_ref[...]


def _tc_main(points, srows, weff, beff, w1, b1):
  return pl.pallas_call(
      _main_body,
      grid=(N // BD,),
      in_specs=[
          pl.BlockSpec((BD, 4), lambda i: (i, 0)),
          pl.BlockSpec((BD, 4), lambda i: (i, 0)),
          pl.BlockSpec((64, 8), lambda i: (0, 0)),
          pl.BlockSpec((1, 64), lambda i: (0, 0)),
          pl.BlockSpec((128, 64), lambda i: (0, 0)),
          pl.BlockSpec((1, 128), lambda i: (0, 0)),
      ],
      out_specs=[
          pl.BlockSpec((BD, 64), lambda i: (i, 0)),
          pl.BlockSpec((BD, 128), lambda i: (i, 0)),
      ],
      out_shape=[
          jax.ShapeDtypeStruct((N, 64), jnp.float32),
          jax.ShapeDtypeStruct((N, 128), jnp.float32),
      ],
      compiler_params=pltpu.CompilerParams(
          dimension_semantics=("arbitrary",)),
  )(points, srows, weff, beff, w1, b1)


def _row_copy(dst_ref, di, src_ref, si):
  for kk in range(8):
    dst_ref[di, pl.ds(kk * 16, 16)] = src_ref[si, pl.ds(kk * 16, 16)]


def _reset_batch(accbuf, fidx, nrows):
  neg = jnp.full((LANES,), NEGINF, jnp.float32)

  def zrow(i, _):
    for kk in range(8):
      accbuf[i, pl.ds(kk * 16, 16)] = neg
    return _

  lax.fori_loop(0, nrows, zrow, 0)
  tr = jnp.full((LANES,), TRASH, jnp.int32)
  for q in range(nrows // 16):
    fidx[pl.ds(q * 16, 16)] = tr


def _sc_segmax_body(h1_hbm, ids_hbm, vmax_hbm, seamr_hbm, seami_hbm,
                    rows_v, ids_v, accbuf, fidx, headrow, tailrow,
                    comb_rows, comb_acc, comb_idx, cids_v, idstage,
                    pair_v, sidebuf, sideids):
  c = lax.axis_index("c")
  s = lax.axis_index("s")
  wid = c * NS + s
  a = wid * P
  lanes = _lanes_i32()

  pltpu.sync_copy(ids_hbm.at[pl.ds(a, 8)], idstage.at[pl.ds(0, 8)])

  @pl.when(wid < NW - 1)
  def _():
    pltpu.sync_copy(ids_hbm.at[pl.ds(a + P, 8)], idstage.at[pl.ds(8, 8)])

  idvec0 = idstage[...]
  first_id = idvec0[0]
  next_first = jnp.where(wid == NW - 1, V, idvec0[8])
  start_row = jnp.where(wid == 0, 0, first_id)

  _reset_batch(accbuf, fidx, FB)

  span = next_first - start_row
  nfull = span // FB
  nrem = span - nfull * FB

  def init_body(i, _):
    pltpu.sync_copy(accbuf, vmax_hbm.at[pl.ds(start_row + i * FB, FB)])
    return _

  lax.fori_loop(0, nfull, init_body, 0)
  rem_base = start_row + nfull * FB

  def init1_body(i, _):
    pltpu.sync_copy(accbuf.at[pl.ds(0, 1)], vmax_hbm.at[pl.ds(rem_base + i, 1)])
    return _

  lax.fori_loop(0, nrem, init1_body, 0)

  def flush(nflush):
    pltpu.sync_copy(accbuf, vmax_hbm.at[fidx])

    @pl.when(nflush == 0)
    def _():
      _row_copy(headrow, 0, accbuf, 0)

    _reset_batch(accbuf, fidx, FB)

  def point(i, carry):
    prev, slot, nflush = carry
    pid = ids_v[pl.ds(i, 16)][0]
    new = pid != prev
    do_flush = jnp.logical_and(new, slot == FB - 1)

    @pl.when(do_flush)
    def _():
      flush(nflush)

    nflush2 = jnp.where(do_flush, nflush + 1, nflush)
    slot2 = jnp.where(new, jnp.where(do_flush, 0, slot + 1), slot)

    @pl.when(new)
    def _():
      is_head = jnp.logical_and(nflush2 == 0, slot2 == 0)
      off = (slot2 // 16) * 16
      vv = fidx[pl.ds(off, 16)]
      val = jnp.where(is_head, TRASH, pid)
      fidx[pl.ds(off, 16)] = jnp.where(lanes + off == slot2, val, vv)

    for kk in range(8):
      av = accbuf[slot2, pl.ds(kk * 16, 16)]
      rv = rows_v[i, pl.ds(kk * 16, 16)]
      accbuf[slot2, pl.ds(kk * 16, 16)] = jnp.maximum(av, rv)

    return pid, slot2, nflush2

  def block(j, carry):
    off = a + j * RB
    pltpu.sync_copy(h1_hbm.at[pl.ds(off, RB)], rows_v)
    pltpu.sync_copy(ids_hbm.at[pl.ds(off, RB)], ids_v.at[pl.ds(0, RB)])
    return lax.fori_loop(0, RB, point, carry)

  prev, slot, nflush = lax.fori_loop(
      0, P // RB, block, (jnp.int32(-1), jnp.int32(-1), jnp.int32(0)))

  _row_copy(tailrow, 0, accbuf, slot)

  @pl.when(nflush == 0)
  def _():
    _row_copy(headrow, 0, accbuf, 0)

  off = (slot // 16) * 16
  vv = fidx[pl.ds(off, 16)]
  fidx[pl.ds(off, 16)] = jnp.where(lanes + off == slot, TRASH, vv)
  pltpu.sync_copy(accbuf, vmax_hbm.at[fidx])

  _row_copy(pair_v, 0, headrow, 0)
  _row_copy(pair_v, 1, tailrow, 0)
  pltpu.sync_copy(pair_v, sidebuf.at[pl.ds(2 * s, 2)])
  idvec = jnp.where(lanes == 0, first_id,
                    jnp.where(lanes == 1, prev, TRASH)).astype(jnp.int32)
  idstage[...] = idvec
  pltpu.sync_copy(idstage, sideids.at[s])
  plsc.subcore_barrier()

  @pl.when(s == 0)
  def _():
    pltpu.sync_copy(sidebuf, comb_rows)
    pltpu.sync_copy(sideids, cids_v)
    _reset_batch(comb_acc, comb_idx, 32)

    def centry(e, carry):
      prev2, slot2 = carry
      crow = cids_v[e // 2, pl.ds(0, 16)]
      eid = jnp.where(e % 2 == 0, crow[0], crow[1])
      new = eid != prev2
      slot3 = jnp.where(new, slot2 + 1, slot2)

      @pl.when(new)
      def _():
        is_excl = jnp.logical_and(c == 1, slot3 == 0)
        off2 = (slot3 // 16) * 16
        vv2 = comb_idx[pl.ds(off2, 16)]
        val2 = jnp.where(is_excl, TRASH, eid)
        comb_idx[pl.ds(off2, 16)] = jnp.where(
            lanes + off2 == slot3, val2, vv2)

      for kk in range(8):
        av = comb_acc[slot3, pl.ds(kk * 16, 16)]
        rv = comb_rows[e, pl.ds(kk * 16, 16)]
        comb_acc[slot3, pl.ds(kk * 16, 16)] = jnp.maximum(av, rv)

      return eid, slot3

    last_id, last_slot = lax.fori_loop(
        0, 2 * NS, centry, (jnp.int32(-1), jnp.int32(-1)))

    seam_slot = jnp.where(c == 0, last_slot, 0)
    seam_id = jnp.where(c == 0, last_id, cids_v[0, pl.ds(0, 16)][0])

    @pl.when(c == 0)
    def _():
      off3 = (last_slot // 16) * 16
      vv3 = comb_idx[pl.ds(off3, 16)]
      comb_idx[pl.ds(off3, 16)] = jnp.where(
          lanes + off3 == last_slot, TRASH, vv3)

    _row_copy(pair_v, 0, comb_acc, seam_slot)
    pltpu.sync_copy(comb_acc, vmax_hbm.at[comb_idx])
    pltpu.sync_copy(pair_v.at[pl.ds(0, 1)], seamr_hbm.at[pl.ds(c, 1)])
    idvec2 = jnp.where(lanes == 0, seam_id, TRASH).astype(jnp.int32)
    idstage[...] = idvec2
    pltpu.sync_copy(idstage.at[pl.ds(0, 8)], seami_hbm.at[c])


def _sc_segmax(h1, inverse_map):
  k = functools.partial(
      pl.kernel,
      out_type=(
          jax.ShapeDtypeStruct((VP, 128), jnp.float32),
          jax.ShapeDtypeStruct((2, 128), jnp.float32),
          jax.ShapeDtypeStruct((2, 8), jnp.int32),
      ),
      mesh=_mesh(),
      compiler_params=pltpu.CompilerParams(use_tc_tiling_on_sc=False),
      scratch_types=[
          pltpu.VMEM((RB, 128), jnp.float32),
          pltpu.VMEM((RB + 16,), jnp.int32),
          pltpu.VMEM((FB, 128), jnp.float32),
          pltpu.VMEM((FB,), jnp.int32),
          pltpu.VMEM((1, 128), jnp.float32),
          pltpu.VMEM((1, 128), jnp.float32),
          pltpu.VMEM((32, 128), jnp.float32),
          pltpu.VMEM((32, 128), jnp.float32),
          pltpu.VMEM((32,), jnp.int32),
          pltpu.VMEM((16, 16), jnp.int32),
          pltpu.VMEM((16,), jnp.int32),
          pltpu.VMEM((2, 128), jnp.float32),
          pltpu.VMEM_SHARED((32, 128), jnp.float32),
          pltpu.VMEM_SHARED((16, 16), jnp.int32),
      ],
  )(_sc_segmax_body)
  return k(h1, inverse_map)


VB = 5_000


def _compress_body(vmax_ref, seamr_ref, seami_ref, wc_ref, bc_ref, out_ref):
  blk = vmax_ref[...]
  seam_rows = seamr_ref[...]
  seam_ids = seami_ref[...]
  vlo = pl.program_id(0) * VB
  rows = lax.broadcasted_iota(jnp.int32, (VB, 1), 0)
  for k in range(2):
    rel = seam_ids[k, 0] - vlo
    blk = jnp.where(rows == rel,
                    jnp.maximum(blk, seam_rows[k:k + 1, :]), blk)
  out = lax.dot_general(blk, wc_ref[...], (((1,), (1,)), ((), ())),
                        preferred_element_type=jnp.float32)
  out_ref[...] = jnp.maximum(out + bc_ref[...], 0.0)


def _tc_compress(vmax_padded, seam_rows, seam_ids, wc, bc):
  return pl.pallas_call(
      _compress_body,
      grid=(V // VB,),
      in_specs=[
          pl.BlockSpec((VB, 128), lambda i: (i, 0)),
          pl.BlockSpec((2, 128), lambda i: (0, 0)),
          pl.BlockSpec((2, 8), lambda i: (0, 0)),
          pl.BlockSpec((16, 128), lambda i: (0, 0)),
          pl.BlockSpec((1, 16), lambda i: (0, 0)),
      ],
      out_specs=pl.BlockSpec((VB, 16), lambda i: (i, 0)),
      out_shape=jax.ShapeDtypeStruct((V, 16), jnp.float32),
  )(vmax_padded, seam_rows, seam_ids, wc, bc)


def kernel(points, inverse_map, voxel_coors, pre_gamma, pre_beta, W0,
           bn0_gamma, bn0_beta, W1, b1, Wc, bc):
  del voxel_coors
  inverse_map = inverse_map.astype(jnp.int32)
  ones_rows = jnp.ones((CH, 4), jnp.float32)
  zeros_rows = jnp.zeros((V, 4), jnp.float32)

  parts = _sc_scatter(points, inverse_map, ones_rows, zeros_rows)
  s_flat = _t1_combine(parts.reshape(4, 4 * V))
  s_table = s_flat.reshape(V, 4)
  srows = _sc_gather(s_table, inverse_map)

  mom = _tc_stats(points, srows)
  sum_f = mom[0]
  m2 = mom[1:9]
  mu = sum_f / N
  cov = m2 / N - mu[:, None] * mu[None, :]
  var_f = jnp.diagonal(cov)
  a = pre_gamma / jnp.sqrt(var_f + EPS)
  cvec = pre_beta - mu * a
  mean0 = W0 @ pre_beta
  aw = W0 * a[None, :]
  var0 = jnp.sum((aw @ cov) * aw, axis=1)
  s0 = bn0_gamma / jnp.sqrt(var0 + EPS)
  t0 = bn0_beta - mean0 * s0
  weff = aw * s0[:, None]
  beff = s0 * (W0 @ cvec) + t0

  h0, h1 = _tc_main(points, srows, weff, beff.reshape(1, 64), W1,
                    b1.reshape(1, 128))

  vmax_padded, seam_rows, seam_ids = _sc_segmax(h1, inverse_map)
  vf = _tc_compress(vmax_padded, seam_rows, seam_ids, Wc, bc.reshape(1, 16))
  return vf, h0, h1
